# baseline (speedup 1.0000x reference)
import jax
import jax.numpy as jnp
from jax.experimental import pallas as pl

_N_E = 8192
_E_DIM = 256
_BETA = 0.25
_TOK_BLK = 256


def _vq_body(z_ref, a_ref, b_ref, emb_ref, zq_ref, idx_ref, loss_ref):
    z = z_ref[...]
    emb = emb_ref[...]
    m = jax.lax.dot_general(z, emb, (((1,), (1,)), ((), ())))
    d = a_ref[...] + b_ref[...] - 2.0 * m
    dmin = jnp.min(d, axis=1, keepdims=True)
    iota = jax.lax.broadcasted_iota(jnp.int32, d.shape, 1)
    idx = jnp.min(jnp.where(d == dmin, iota, jnp.int32(_N_E)),
                  axis=1, keepdims=True)
    oh = (iota == idx).astype(jnp.float32)
    zq = jax.lax.dot_general(oh, emb, (((1,), (0,)), ((), ())))
    zq_ref[...] = z + (zq - z)
    idx_ref[...] = idx.astype(jnp.int32)

    @pl.when(pl.program_id(0) == 0)
    def _init():
        loss_ref[...] = jnp.zeros_like(loss_ref)

    loss_ref[...] += jnp.sum(dmin, keepdims=True)


def kernel(z, emb_weight):
    B, C, H, W = z.shape
    z_p = jnp.transpose(z, (0, 2, 3, 1))
    z_flat = z_p.reshape(-1, _E_DIM)
    n_tok = z_flat.shape[0]
    a = jnp.sum(z_flat ** 2, axis=1, keepdims=True)
    b = jnp.sum(emb_weight ** 2, axis=1)[None, :]

    grid = (n_tok // _TOK_BLK,)
    zq_flat, idx2, loss_sum = pl.pallas_call(
        _vq_body,
        grid=grid,
        in_specs=[
            pl.BlockSpec((_TOK_BLK, _E_DIM), lambda i: (i, 0)),
            pl.BlockSpec((_TOK_BLK, 1), lambda i: (i, 0)),
            pl.BlockSpec((1, _N_E), lambda i: (0, 0)),
            pl.BlockSpec((_N_E, _E_DIM), lambda i: (0, 0)),
        ],
        out_specs=[
            pl.BlockSpec((_TOK_BLK, _E_DIM), lambda i: (i, 0)),
            pl.BlockSpec((_TOK_BLK, 1), lambda i: (i, 0)),
            pl.BlockSpec((1, 1), lambda i: (0, 0)),
        ],
        out_shape=[
            jax.ShapeDtypeStruct((n_tok, _E_DIM), jnp.float32),
            jax.ShapeDtypeStruct((n_tok, 1), jnp.int32),
            jax.ShapeDtypeStruct((1, 1), jnp.float32),
        ],
    )(z_flat, a, b, emb_weight)

    n_el = jnp.float32(n_tok * _E_DIM)
    s = loss_sum[0, 0]
    loss = s / n_el + _BETA * (s / n_el)
    z_q_out = jnp.transpose(zq_flat.reshape(B, H, W, C), (0, 3, 1, 2))
    return (z_q_out, loss, idx2.reshape(-1))


# speedup vs baseline: 1.7017x; 1.7017x over previous
import jax
import jax.numpy as jnp
from jax.experimental import pallas as pl
from jax.experimental.pallas import tpu as pltpu
from jax.experimental.pallas import tpu_sc as plsc

_N_E = 8192
_E_DIM = 256
_BETA = 0.25
_TOK_BLK = 256
_GATHER_BLK = 128


def _dist_body(z_ref, a_ref, b_ref, emb_ref, idx_ref, part_ref):
    z = z_ref[...]
    emb = emb_ref[...]
    m = jax.lax.dot_general(z, emb, (((1,), (1,)), ((), ())))
    d = a_ref[...] + b_ref[...] - 2.0 * m
    dmin = jnp.min(d, axis=1, keepdims=True)
    iota = jax.lax.broadcasted_iota(jnp.int32, d.shape, 1)
    idx = jnp.min(jnp.where(d == dmin, iota, jnp.int32(_N_E)),
                  axis=1, keepdims=True)
    idx_ref[...] = idx
    part_ref[...] = jnp.sum(dmin, keepdims=True).reshape(1, 1, 1)


def _distance_argmin(z_flat, a, b, emb_weight):
    n_tok = z_flat.shape[0]
    grid = (n_tok // _TOK_BLK,)
    return pl.pallas_call(
        _dist_body,
        grid=grid,
        in_specs=[
            pl.BlockSpec((_TOK_BLK, _E_DIM), lambda i: (i, 0)),
            pl.BlockSpec((_TOK_BLK, 1), lambda i: (i, 0)),
            pl.BlockSpec((1, _N_E), lambda i: (0, 0)),
            pl.BlockSpec((_N_E, _E_DIM), lambda i: (0, 0)),
        ],
        out_specs=[
            pl.BlockSpec((_TOK_BLK, 1), lambda i: (i, 0)),
            pl.BlockSpec((1, 1, 1), lambda i: (i, 0, 0)),
        ],
        out_shape=[
            jax.ShapeDtypeStruct((n_tok, 1), jnp.int32),
            jax.ShapeDtypeStruct((grid[0], 1, 1), jnp.float32),
        ],
    )(z_flat, a, b, emb_weight)


def _sc_gather(emb_weight, idx_row, n_tok):
    mesh = plsc.VectorSubcoreMesh(core_axis_name="c", subcore_axis_name="s")

    @pl.kernel(
        out_type=jax.ShapeDtypeStruct((n_tok, _E_DIM), jnp.float32),
        mesh=mesh,
    )
    def gather_kernel(emb_hbm, i_hbm, o_hbm):
        def body(i_vmem, o_vmem):
            pltpu.sync_copy(emb_hbm.at[i_vmem.at[0]], o_vmem)

        pltpu.emit_pipeline(
            body,
            grid=(n_tok // _GATHER_BLK,),
            in_specs=[pl.BlockSpec((1, _GATHER_BLK), index_map=lambda i: (0, i))],
            out_specs=[pl.BlockSpec((_GATHER_BLK, _E_DIM),
                                    index_map=lambda i: (i, 0))],
            core_axis_name=("c", "s"),
            dimension_semantics=(pltpu.PARALLEL,),
        )(i_hbm, o_hbm)

    return gather_kernel(emb_weight, idx_row)


def kernel(z, emb_weight):
    B, C, H, W = z.shape
    z_p = jnp.transpose(z, (0, 2, 3, 1))
    z_flat = z_p.reshape(-1, _E_DIM)
    n_tok = z_flat.shape[0]
    a = jnp.sum(z_flat ** 2, axis=1, keepdims=True)
    b = jnp.sum(emb_weight ** 2, axis=1)[None, :]

    idx2, parts = _distance_argmin(z_flat, a, b, emb_weight)
    zq_flat = _sc_gather(emb_weight, idx2.reshape(1, -1), n_tok)

    n_el = jnp.float32(n_tok * _E_DIM)
    s = jnp.sum(parts)
    loss = s / n_el + _BETA * (s / n_el)
    z_q_out = jnp.transpose(zq_flat.reshape(B, H, W, C), (0, 3, 1, 2))
    return (z_q_out, loss, idx2.reshape(-1))
